# baseline (speedup 1.0000x reference)
import functools

import jax
import jax.numpy as jnp
from jax import lax
from jax.experimental import pallas as pl
from jax.experimental.pallas import tpu as pltpu
from jax.experimental.pallas import tpu_sc as plsc

N_NODES = 10000
N1 = 2500
N2 = 800
E0 = 20000
E1 = 6000
D_IN = 128
D_EDGE = 16
EMB = 64
HID = 64
OUT = 8

_NC = 2
_NS = 16
_NW = _NC * _NS

_F32 = jnp.float32


def _leaky(x):
    return jnp.where(x >= 0, x, 0.01 * x)


def _ln(x, g, b, eps=1e-5):
    mu = jnp.mean(x, axis=-1, keepdims=True)
    var = jnp.mean((x - mu) ** 2, axis=-1, keepdims=True)
    return (x - mu) / jnp.sqrt(var + eps) * g + b


def _bn(x, g, b, eps=1e-5):
    mu = jnp.mean(x, axis=0, keepdims=True)
    var = jnp.mean((x - mu) ** 2, axis=0, keepdims=True)
    return (x - mu) / jnp.sqrt(var + eps) * g + b


def _dot(a, b):
    return jnp.dot(a, b, preferred_element_type=_F32)


def _mlp3_body(x_ref, w1, b1, w2, b2, w3, b3, o_ref, *, final_act):
    h = _leaky(_dot(x_ref[...], w1[...]) + b1[...])
    h = _leaky(_dot(h, w2[...]) + b2[...])
    o = _dot(h, w3[...]) + b3[...]
    if final_act:
        o = _leaky(o)
    o_ref[...] = o


def _run_mlp3_tc(x, ps, final_act, tile_rows):
    n, d = x.shape
    assert n % tile_rows == 0
    (w1, b1), (w2, b2), (w3, b3) = ps
    grid = n // tile_rows
    full = lambda a: pl.BlockSpec(a.shape, lambda i: (0,) * a.ndim)
    args = (w1, b1.reshape(1, -1), w2, b2.reshape(1, -1), w3, b3.reshape(1, -1))
    return pl.pallas_call(
        functools.partial(_mlp3_body, final_act=final_act),
        grid=(grid,),
        in_specs=[pl.BlockSpec((tile_rows, d), lambda i: (i, 0))]
        + [full(a) for a in args],
        out_specs=pl.BlockSpec((tile_rows, w3.shape[1]), lambda i: (i, 0)),
        out_shape=jax.ShapeDtypeStruct((n, w3.shape[1]), _F32),
    )(x, *args)


def _edge_pre_body(ea_ref, w1, b1, w2, b2, o_ref):
    h = _leaky(_dot(ea_ref[...], w1[...]) + b1[...])
    o_ref[...] = _leaky(_dot(h, w2[...]) + b2[...])


def _run_edge_pre_tc(ea_p, ps, te=512):
    ep = ea_p.shape[0]
    (w1, b1), (w2, b2), _ = ps
    full = lambda a: pl.BlockSpec(a.shape, lambda i: (0,) * a.ndim)
    args = (w1, b1.reshape(1, -1), w2, b2.reshape(1, -1))
    return pl.pallas_call(
        _edge_pre_body,
        grid=(ep // te,),
        in_specs=[pl.BlockSpec((te, D_EDGE), lambda i: (i, 0))]
        + [full(a) for a in args],
        out_specs=pl.BlockSpec((te, 128), lambda i: (i, 0)),
        out_shape=jax.ShapeDtypeStruct((ep, 128), _F32),
    )(ea_p, *args)


def _edge_body(h2_ref, xj_ref, rex_ref, w3, b3, o_ref, *, n_real, te, in_ch):
    pid = pl.program_id(0)
    wf = _dot(h2_ref[...], w3[...]) + b3[...]
    wf = wf.astype(jnp.bfloat16).astype(_F32)
    xj = xj_ref[...].astype(jnp.bfloat16).astype(_F32)
    x_rep = _dot(xj[:, :64], rex_ref[...])
    prod = wf * x_rep
    msg = jnp.zeros((te, 64), _F32)
    for i2 in range(in_ch // 2):
        q = prod[:, i2 * 128:(i2 + 1) * 128]
        msg = (msg + q[:, :64]) + q[:, 64:]
    row = pid * te + lax.broadcasted_iota(jnp.int32, (te, 1), 0)
    valid = row < n_real
    msg = jnp.where(valid, msg, 0.0)
    flag = jnp.where(valid, 1.0, 0.0).astype(_F32)
    o_ref[...] = jnp.concatenate([msg, flag, jnp.zeros((te, 63), _F32)], axis=1)


def _run_edge_tc(h2_p, xj_p, ps, n_real, te=256):
    ep = h2_p.shape[0]
    assert ep % te == 0
    _, _, (w3, b3) = ps
    in_ch = w3.shape[1] // 64
    rex = (jnp.arange(in_ch * 64)[None, :] // 64
           == jnp.arange(in_ch)[:, None]).astype(_F32)
    full = lambda a: pl.BlockSpec(a.shape, lambda i: (0,) * a.ndim)
    args = (rex, w3, b3.reshape(1, -1))
    return pl.pallas_call(
        functools.partial(_edge_body, n_real=n_real, te=te, in_ch=in_ch),
        grid=(ep // te,),
        in_specs=[
            pl.BlockSpec((te, 128), lambda i: (i, 0)),
            pl.BlockSpec((te, 128), lambda i: (i, 0)),
        ] + [full(a) for a in args],
        out_specs=pl.BlockSpec((te, 128), lambda i: (i, 0)),
        out_shape=jax.ShapeDtypeStruct((ep, 128), _F32),
    )(h2_p, xj_p, *args)


def _sc_gather(table, idx3, epw):
    nw, c_chunks, chunk = idx3.shape
    n_out = nw * epw
    d = table.shape[1]
    mesh = plsc.VectorSubcoreMesh(core_axis_name="c", subcore_axis_name="s",
                                  num_cores=_NC, num_subcores=_NS)

    @functools.partial(
        pl.kernel,
        out_type=jax.ShapeDtypeStruct((n_out, d), _F32),
        mesh=mesh,
        scratch_types=[
            pltpu.VMEM((c_chunks, chunk), jnp.int32),
            pltpu.VMEM((epw, d), _F32),
            pltpu.SemaphoreType.DMA,
        ],
    )
    def k(tab_hbm, idx_hbm, out_hbm, idx_v, rows_v, sem):
        wid = lax.axis_index("s") * _NC + lax.axis_index("c")
        pltpu.sync_copy(idx_hbm.at[wid], idx_v)
        descs = [
            pltpu.async_copy(
                tab_hbm.at[idx_v.at[j]], rows_v.at[pl.ds(j * chunk, chunk)], sem)
            for j in range(c_chunks)
        ]
        for dsc in descs:
            dsc.wait()
        pltpu.sync_copy(rows_v, out_hbm.at[pl.ds(wid * epw, epw)])

    return k(table, idx3)


def _sc_scatter(msg_p, dst2, n_seg, win=256):
    ep = msg_p.shape[0]
    nwin = ep // win
    cpw = win // 128
    zeros = jnp.zeros((n_seg, 128), _F32)
    mesh = plsc.VectorSubcoreMesh(core_axis_name="c", subcore_axis_name="s",
                                  num_cores=_NC, num_subcores=_NS)

    @functools.partial(
        pl.kernel,
        out_type=jax.ShapeDtypeStruct((n_seg, 128), _F32),
        mesh=mesh,
        scratch_types=[
            pltpu.VMEM((ep // 128, 128), jnp.int32),
            pltpu.VMEM((2, win, 128), _F32),
            pltpu.VMEM_SHARED((n_seg, 128), _F32),
            pltpu.SemaphoreType.DMA,
            pltpu.SemaphoreType.DMA,
        ],
    )
    def k(msg_hbm, dst_hbm, z_hbm, out_hbm, idx_v, msg_v, acc_sh, sem_l, sem_a):
        c = lax.axis_index("c")
        s = lax.axis_index("s")

        @pl.when(jnp.logical_and(c == 0, s == 0))
        def _():
            pltpu.sync_copy(z_hbm, acc_sh)
            pltpu.sync_copy(dst_hbm, idx_v)
            ld = pltpu.async_copy(msg_hbm.at[pl.ds(0, win)], msg_v.at[0], sem_l)
            pend = []
            for w in range(nwin):
                b = w % 2
                nld = None
                if w + 1 < nwin:
                    if pend:
                        for dsc in pend.pop(0):
                            dsc.wait()
                    nld = pltpu.async_copy(
                        msg_hbm.at[pl.ds((w + 1) * win, win)], msg_v.at[1 - b], sem_l)
                ld.wait()
                pend.append([
                    pltpu.async_copy(msg_v.at[b].at[pl.ds(j * 128, 128)],
                                     acc_sh.at[idx_v.at[w * cpw + j]], sem_a,
                                     add=True)
                    for j in range(cpw)])
                ld = nld
            for descs in pend:
                for dsc in descs:
                    dsc.wait()
            pltpu.sync_copy(acc_sh, out_hbm)

    return k(msg_p, dst2, zeros)


def _pad_rows(a, n):
    return jnp.pad(a, ((0, n - a.shape[0]),) + ((0, 0),) * (a.ndim - 1))


def _pad_last_mlp(ps):
    (w1, b1), (w2, b2), (w3, b3) = ps
    w3p = jnp.pad(w3, ((0, 0), (0, 64)))
    b3p = jnp.pad(b3, (0, 64))
    return [(w1, b1), (w2, b2), (w3p, b3p)]


def kernel(x_target, x_entity, src0, dst0, edge_attr0, src1, dst1, edge_attr1, params):
    h_t = _run_mlp3_tc(x_target, _pad_last_mlp(params["emb_target"]), True, 400)
    h_e = _run_mlp3_tc(x_entity, _pad_last_mlp(params["emb_entity"]), True, 400)
    h = jnp.concatenate([h_t, h_e], axis=0)

    h2_0 = _run_edge_pre_tc(_pad_rows(edge_attr0, 20480), params["conv0"]["edge_nn"])
    h2_1 = _run_edge_pre_tc(_pad_rows(edge_attr1, 8192), params["conv1"]["edge_nn"])

    def conv(h_tab, src, dst, h2, conv_p, bn_p, n_tgt, n_pad, ep, c_chunks, n_real):
        idx = _pad_rows(src[:, None], ep).reshape(_NW, c_chunks, 128)
        xj = _sc_gather(h_tab, idx, c_chunks * 128)
        msg = _run_edge_tc(h2, xj, conv_p["edge_nn"], n_real)
        dst2 = _pad_rows(dst[:, None], ep).reshape(-1, 128)
        part = _sc_scatter(msg, dst2, n_pad)
        s = part[:n_tgt, :64]
        deg = part[:n_tgt, 64:65]
        agg = s / jnp.clip(deg, 1.0)
        x = jnp.concatenate([_ln(agg, *conv_p["ln1"]),
                             _ln(h_tab[:n_tgt, :64], *conv_p["ln2"])], axis=1)
        rows = ((n_tgt + 511) // 512) * 512
        x = _run_mlp3_tc(_pad_rows(x, rows), conv_p["node_nn"], True, 512)[:n_tgt]
        return _bn(x, *bn_p)

    h1 = conv(h, src0, dst0, h2_0, params["conv0"], params["bn0"],
              N1, 2560, 20480, 5, E0)
    h1p = jnp.concatenate([h1, jnp.zeros((N1, 64), _F32)], axis=1)
    hh = conv(h1p, src1, dst1, h2_1, params["conv1"], params["bn1"],
              N2, 896, 8192, 2, E1)
    rows2 = 832
    out = _run_mlp3_tc(_pad_rows(hh, rows2), params["out_nn"], False, 416)[:N2]
    return out


# speedup vs baseline: 1.1868x; 1.1868x over previous
import functools

import jax
import jax.numpy as jnp
from jax import lax
from jax.experimental import pallas as pl
from jax.experimental.pallas import tpu as pltpu
from jax.experimental.pallas import tpu_sc as plsc

N_NODES = 10000
N1 = 2500
N2 = 800
E0 = 20000
E1 = 6000
D_IN = 128
D_EDGE = 16
EMB = 64
HID = 64
OUT = 8

_NC = 2
_NS = 16
_NW = _NC * _NS

_F32 = jnp.float32


def _leaky(x):
    return jnp.where(x >= 0, x, 0.01 * x)


def _ln(x, g, b, eps=1e-5):
    mu = jnp.mean(x, axis=-1, keepdims=True)
    var = jnp.mean((x - mu) ** 2, axis=-1, keepdims=True)
    return (x - mu) / jnp.sqrt(var + eps) * g + b


def _bn(x, g, b, eps=1e-5):
    mu = jnp.mean(x, axis=0, keepdims=True)
    var = jnp.mean((x - mu) ** 2, axis=0, keepdims=True)
    return (x - mu) / jnp.sqrt(var + eps) * g + b


def _dot(a, b):
    return jnp.dot(a, b, preferred_element_type=_F32)


def _mlp3_body(x_ref, w1, b1, w2, b2, w3, b3, o_ref, *, final_act):
    h = _leaky(_dot(x_ref[...], w1[...]) + b1[...])
    h = _leaky(_dot(h, w2[...]) + b2[...])
    o = _dot(h, w3[...]) + b3[...]
    if final_act:
        o = _leaky(o)
    o_ref[...] = o


def _run_mlp3_tc(x, ps, final_act, tile_rows):
    n, d = x.shape
    assert n % tile_rows == 0
    (w1, b1), (w2, b2), (w3, b3) = ps
    grid = n // tile_rows
    full = lambda a: pl.BlockSpec(a.shape, lambda i: (0,) * a.ndim)
    args = (w1, b1.reshape(1, -1), w2, b2.reshape(1, -1), w3, b3.reshape(1, -1))
    return pl.pallas_call(
        functools.partial(_mlp3_body, final_act=final_act),
        grid=(grid,),
        in_specs=[pl.BlockSpec((tile_rows, d), lambda i: (i, 0))]
        + [full(a) for a in args],
        out_specs=pl.BlockSpec((tile_rows, w3.shape[1]), lambda i: (i, 0)),
        out_shape=jax.ShapeDtypeStruct((n, w3.shape[1]), _F32),
    )(x, *args)


def _edge_body(ea_ref, xj_ref, rex_ref, w1, b1, w2, b2, w3, b3, o_ref, *,
               n_real, te, in_ch):
    pid = pl.program_id(0)
    h = _leaky(_dot(ea_ref[...], w1[...]) + b1[...])
    h = _leaky(_dot(h, w2[...]) + b2[...])
    wf = _dot(h, w3[...]) + b3[...]
    wf = wf.astype(jnp.bfloat16).astype(_F32)
    xj = xj_ref[...].astype(jnp.bfloat16).astype(_F32)
    x_rep = _dot(xj[:, :64], rex_ref[...])
    prod = wf * x_rep
    msg = jnp.zeros((te, 64), _F32)
    for i2 in range(in_ch // 2):
        q = prod[:, i2 * 128:(i2 + 1) * 128]
        msg = (msg + q[:, :64]) + q[:, 64:]
    row = pid * te + lax.broadcasted_iota(jnp.int32, (te, 1), 0)
    valid = row < n_real
    msg = jnp.where(valid, msg, 0.0)
    flag = jnp.where(valid, 1.0, 0.0).astype(_F32)
    o_ref[...] = jnp.concatenate([msg, flag, jnp.zeros((te, 63), _F32)], axis=1)


def _run_edge_tc(ea_p, xj_p, ps, n_real, te=256):
    ep = ea_p.shape[0]
    assert ep % te == 0
    (w1, b1), (w2, b2), (w3, b3) = ps
    in_ch = w3.shape[1] // 64
    rex = (jnp.arange(in_ch * 64)[None, :] // 64
           == jnp.arange(in_ch)[:, None]).astype(_F32)
    full = lambda a: pl.BlockSpec(a.shape, lambda i: (0,) * a.ndim)
    args = (rex, w1, b1.reshape(1, -1), w2, b2.reshape(1, -1), w3, b3.reshape(1, -1))
    return pl.pallas_call(
        functools.partial(_edge_body, n_real=n_real, te=te, in_ch=in_ch),
        grid=(ep // te,),
        in_specs=[
            pl.BlockSpec((te, D_EDGE), lambda i: (i, 0)),
            pl.BlockSpec((te, 128), lambda i: (i, 0)),
        ] + [full(a) for a in args],
        out_specs=pl.BlockSpec((te, 128), lambda i: (i, 0)),
        out_shape=jax.ShapeDtypeStruct((ep, 128), _F32),
    )(ea_p, xj_p, *args)


def _sc_gather(table, idx3, epw):
    nw, c_chunks, chunk = idx3.shape
    n_out = nw * epw
    d = table.shape[1]
    mesh = plsc.VectorSubcoreMesh(core_axis_name="c", subcore_axis_name="s",
                                  num_cores=_NC, num_subcores=_NS)

    @functools.partial(
        pl.kernel,
        out_type=jax.ShapeDtypeStruct((n_out, d), _F32),
        mesh=mesh,
        scratch_types=[
            pltpu.VMEM((c_chunks, chunk), jnp.int32),
            pltpu.VMEM((epw, d), _F32),
            pltpu.SemaphoreType.DMA,
        ],
    )
    def k(tab_hbm, idx_hbm, out_hbm, idx_v, rows_v, sem):
        wid = lax.axis_index("s") * _NC + lax.axis_index("c")
        pltpu.sync_copy(idx_hbm.at[wid], idx_v)
        descs = [
            pltpu.async_copy(
                tab_hbm.at[idx_v.at[j]], rows_v.at[pl.ds(j * chunk, chunk)], sem)
            for j in range(c_chunks)
        ]
        for dsc in descs:
            dsc.wait()
        pltpu.sync_copy(rows_v, out_hbm.at[pl.ds(wid * epw, epw)])

    return k(table, idx3)


def _sc_scatter(msg_p, dst2, n_seg, win=256):
    ep = msg_p.shape[0]
    nwin = ep // win
    cpw = win // 128
    zeros = jnp.zeros((n_seg, 128), _F32)
    mesh = plsc.VectorSubcoreMesh(core_axis_name="c", subcore_axis_name="s",
                                  num_cores=_NC, num_subcores=_NS)

    @functools.partial(
        pl.kernel,
        out_type=jax.ShapeDtypeStruct((n_seg, 128), _F32),
        mesh=mesh,
        scratch_types=[
            pltpu.VMEM((ep // 128, 128), jnp.int32),
            pltpu.VMEM((2, win, 128), _F32),
            pltpu.VMEM_SHARED((n_seg, 128), _F32),
            pltpu.SemaphoreType.DMA,
            pltpu.SemaphoreType.DMA,
        ],
    )
    def k(msg_hbm, dst_hbm, z_hbm, out_hbm, idx_v, msg_v, acc_sh, sem_l, sem_a):
        c = lax.axis_index("c")
        s = lax.axis_index("s")

        @pl.when(jnp.logical_and(c == 0, s == 0))
        def _():
            pltpu.sync_copy(z_hbm, acc_sh)
            pltpu.sync_copy(dst_hbm, idx_v)
            ld = pltpu.async_copy(msg_hbm.at[pl.ds(0, win)], msg_v.at[0], sem_l)
            pend = []
            for w in range(nwin):
                b = w % 2
                nld = None
                if w + 1 < nwin:
                    if pend:
                        for dsc in pend.pop(0):
                            dsc.wait()
                    nld = pltpu.async_copy(
                        msg_hbm.at[pl.ds((w + 1) * win, win)], msg_v.at[1 - b], sem_l)
                ld.wait()
                pend.append([
                    pltpu.async_copy(msg_v.at[b].at[pl.ds(j * 128, 128)],
                                     acc_sh.at[idx_v.at[w * cpw + j]], sem_a,
                                     add=True)
                    for j in range(cpw)])
                ld = nld
            for descs in pend:
                for dsc in descs:
                    dsc.wait()
            pltpu.sync_copy(acc_sh, out_hbm)

    return k(msg_p, dst2, zeros)


def _pad_rows(a, n):
    return jnp.pad(a, ((0, n - a.shape[0]),) + ((0, 0),) * (a.ndim - 1))


def _pad_last_mlp(ps):
    (w1, b1), (w2, b2), (w3, b3) = ps
    w3p = jnp.pad(w3, ((0, 0), (0, 64)))
    b3p = jnp.pad(b3, (0, 64))
    return [(w1, b1), (w2, b2), (w3p, b3p)]


def kernel(x_target, x_entity, src0, dst0, edge_attr0, src1, dst1, edge_attr1, params):
    h_t = _run_mlp3_tc(x_target, _pad_last_mlp(params["emb_target"]), True, 400)
    h_e = _run_mlp3_tc(x_entity, _pad_last_mlp(params["emb_entity"]), True, 400)
    h = jnp.concatenate([h_t, h_e], axis=0)

    def conv(h_tab, src, dst, ea, conv_p, bn_p, n_tgt, n_pad, ep, c_chunks,
             chunk, n_real):
        idx = _pad_rows(src[:, None], ep).reshape(_NW, c_chunks, chunk)
        xj = _sc_gather(h_tab, idx, c_chunks * chunk)
        msg = _run_edge_tc(_pad_rows(ea, ep), xj, conv_p["edge_nn"], n_real)
        dst2 = _pad_rows(dst[:, None], ep).reshape(-1, 128)
        part = _sc_scatter(msg, dst2, n_pad)
        s = part[:n_tgt, :64]
        deg = part[:n_tgt, 64:65]
        agg = s / jnp.clip(deg, 1.0)
        x = jnp.concatenate([_ln(agg, *conv_p["ln1"]),
                             _ln(h_tab[:n_tgt, :64], *conv_p["ln2"])], axis=1)
        rows = ((n_tgt + 511) // 512) * 512
        x = _run_mlp3_tc(_pad_rows(x, rows), conv_p["node_nn"], True, 512)[:n_tgt]
        return _bn(x, *bn_p)

    h1 = conv(h, src0, dst0, edge_attr0, params["conv0"], params["bn0"],
              N1, 2560, 20480, 5, 128, E0)
    h1p = jnp.concatenate([h1, jnp.zeros((N1, 64), _F32)], axis=1)
    hh = conv(h1p, src1, dst1, edge_attr1, params["conv1"], params["bn1"],
              N2, 896, 6400, 2, 100, E1)
    rows2 = 832
    out = _run_mlp3_tc(_pad_rows(hh, rows2), params["out_nn"], False, 416)[:N2]
    return out
